# baseline (speedup 1.0000x reference)
import jax
import jax.numpy as jnp
from jax.experimental import pallas as pl
from jax.experimental.pallas import tpu as pltpu

_CR = 256
_NR = 4
_C = 1024
_NBUF = 5


def _ld(x_hbm, x_bufs, sems, j, s, c, hbase, sbase):
    return pltpu.make_async_copy(
        x_hbm.at[pl.ds(hbase + j * c, c), :],
        x_bufs.at[pl.ds(sbase + s * c, c), :],
        sems.at[s],
    )


def _st(o_bufs, out_hbm, sems, j, s, c, hbase, sbase):
    return pltpu.make_async_copy(
        o_bufs.at[pl.ds(sbase + s * c, c), :],
        out_hbm.at[pl.ds(hbase + j * c, c), :],
        sems.at[s],
    )


def _body(idx_hbm, w_hbm, x_hbm, out_hbm, x_bufs, o_bufs, idx_ref, w_ref,
          rl_sems, rs_sems, load_sems, store_sems, small_sem):
    n_rows = x_hbm.shape[0]
    ramp_rows = _NR * _CR
    n_chunks = (n_rows - ramp_rows) // _C
    rbase = _NBUF * _C

    idx_cp = pltpu.make_async_copy(idx_hbm, idx_ref, small_sem)
    w_cp = pltpu.make_async_copy(w_hbm, w_ref, small_sem)
    idx_cp.start()
    w_cp.start()
    for k in range(_NR):
        _ld(x_hbm, x_bufs, rl_sems, k, k, _CR, 0, rbase).start()
    for j in range(_NBUF):
        _ld(x_hbm, x_bufs, load_sems, j, j, _C, ramp_rows, 0).start()
    idx_cp.wait()
    w_cp.wait()

    for k in range(_NR):
        _ld(x_hbm, x_bufs, rl_sems, k, k, _CR, 0, rbase).wait()
        scale = idx_ref[pl.ds(k * _CR, _CR)].astype(jnp.float32)[:, None]
        o_bufs[pl.ds(rbase + k * _CR, _CR), :] = (
            x_bufs[pl.ds(rbase + k * _CR, _CR), :] + scale * w_ref[...])
        _st(o_bufs, out_hbm, rs_sems, k, k, _CR, 0, rbase).start()

    def step(i, carry):
        s = jax.lax.rem(i, _NBUF)
        _ld(x_hbm, x_bufs, load_sems, i, s, _C, ramp_rows, 0).wait()

        @pl.when(i >= _NBUF)
        def _():
            _st(o_bufs, out_hbm, store_sems, i - _NBUF, s, _C, ramp_rows,
                0).wait()

        scale = idx_ref[pl.ds(ramp_rows + i * _C, _C)].astype(
            jnp.float32)[:, None]
        o_bufs[pl.ds(s * _C, _C), :] = (
            x_bufs[pl.ds(s * _C, _C), :] + scale * w_ref[...])

        @pl.when(i + _NBUF < n_chunks)
        def _():
            _ld(x_hbm, x_bufs, load_sems, i + _NBUF, s, _C, ramp_rows,
                0).start()

        _st(o_bufs, out_hbm, store_sems, i, s, _C, ramp_rows, 0).start()
        return carry

    jax.lax.fori_loop(0, n_chunks, step, 0)

    for k in range(_NR):
        _st(o_bufs, out_hbm, rs_sems, k, k, _CR, 0, rbase).wait()
    for j in range(n_chunks - _NBUF, n_chunks):
        _st(o_bufs, out_hbm, store_sems, j, j % _NBUF, _C, ramp_rows,
            0).wait()


def kernel(x, indices, embs_weight):
    B, S, D = x.shape
    n_rows = B * S
    x2 = x.reshape(n_rows, D)
    idx1 = indices.reshape(n_rows)
    buf_rows = _NBUF * _C + _NR * _CR
    out = pl.pallas_call(
        _body,
        in_specs=[
            pl.BlockSpec(memory_space=pl.ANY),
            pl.BlockSpec(memory_space=pl.ANY),
            pl.BlockSpec(memory_space=pl.ANY),
        ],
        out_specs=pl.BlockSpec(memory_space=pl.ANY),
        out_shape=jax.ShapeDtypeStruct((n_rows, D), x.dtype),
        scratch_shapes=[
            pltpu.VMEM((buf_rows, D), jnp.float32),
            pltpu.VMEM((buf_rows, D), jnp.float32),
            pltpu.VMEM((n_rows,), indices.dtype),
            pltpu.VMEM((1, D), jnp.float32),
            pltpu.SemaphoreType.DMA((_NR,)),
            pltpu.SemaphoreType.DMA((_NR,)),
            pltpu.SemaphoreType.DMA((_NBUF,)),
            pltpu.SemaphoreType.DMA((_NBUF,)),
            pltpu.SemaphoreType.DMA,
        ],
    )(idx1, embs_weight, x2)
    return out.reshape(B, S, D)


# speedup vs baseline: 1.0034x; 1.0034x over previous
import jax
import jax.numpy as jnp
from jax.experimental import pallas as pl
from jax.experimental.pallas import tpu as pltpu

_C = 1024
_NBUF = 6


def _load(x_hbm, x_bufs, load_sems, j, s):
    return pltpu.make_async_copy(
        x_hbm.at[pl.ds(j * _C, _C), :],
        x_bufs.at[pl.ds(s * _C, _C), :],
        load_sems.at[s],
    )


def _store(o_bufs, out_hbm, store_sems, j, s):
    return pltpu.make_async_copy(
        o_bufs.at[pl.ds(s * _C, _C), :],
        out_hbm.at[pl.ds(j * _C, _C), :],
        store_sems.at[s],
    )


def _body(idx_hbm, w_hbm, x_hbm, out_hbm, x_bufs, o_bufs, idx_ref, w_ref,
          load_sems, store_sems, small_sem):
    n_rows = x_hbm.shape[0]
    n_chunks = n_rows // _C

    idx_cp = pltpu.make_async_copy(idx_hbm, idx_ref, small_sem)
    w_cp = pltpu.make_async_copy(w_hbm, w_ref, small_sem)
    idx_cp.start()
    w_cp.start()
    for j in range(_NBUF):
        _load(x_hbm, x_bufs, load_sems, j, j).start()
    idx_cp.wait()
    w_cp.wait()

    def step(i, carry):
        s = jax.lax.rem(i, _NBUF)
        _load(x_hbm, x_bufs, load_sems, i, s).wait()

        @pl.when(i >= _NBUF)
        def _():
            _store(o_bufs, out_hbm, store_sems, i - _NBUF, s).wait()

        scale = idx_ref[pl.ds(i, 1), :][0, :].astype(jnp.float32)[:, None]
        o_bufs[pl.ds(s * _C, _C), :] = (
            x_bufs[pl.ds(s * _C, _C), :] + scale * w_ref[...])

        @pl.when(i + _NBUF < n_chunks)
        def _():
            _load(x_hbm, x_bufs, load_sems, i + _NBUF, s).start()

        _store(o_bufs, out_hbm, store_sems, i, s).start()
        return carry

    jax.lax.fori_loop(0, n_chunks, step, 0)

    for j in range(n_chunks - _NBUF, n_chunks):
        _store(o_bufs, out_hbm, store_sems, j, j % _NBUF).wait()


def kernel(x, indices, embs_weight):
    B, S, D = x.shape
    n_rows = B * S
    n_chunks = n_rows // _C
    x2 = x.reshape(n_rows, D)
    idx2 = indices.reshape(n_chunks, _C)
    out = pl.pallas_call(
        _body,
        in_specs=[
            pl.BlockSpec(memory_space=pl.ANY),
            pl.BlockSpec(memory_space=pl.ANY),
            pl.BlockSpec(memory_space=pl.ANY),
        ],
        out_specs=pl.BlockSpec(memory_space=pl.ANY),
        out_shape=jax.ShapeDtypeStruct((n_rows, D), x.dtype),
        scratch_shapes=[
            pltpu.VMEM((_NBUF * _C, D), jnp.float32),
            pltpu.VMEM((_NBUF * _C, D), jnp.float32),
            pltpu.VMEM((n_chunks, _C), indices.dtype),
            pltpu.VMEM((1, D), jnp.float32),
            pltpu.SemaphoreType.DMA((_NBUF,)),
            pltpu.SemaphoreType.DMA((_NBUF,)),
            pltpu.SemaphoreType.DMA,
        ],
    )(idx2, embs_weight, x2)
    return out.reshape(B, S, D)


# speedup vs baseline: 1.0100x; 1.0066x over previous
import jax
import jax.numpy as jnp
from jax.experimental import pallas as pl
from jax.experimental.pallas import tpu as pltpu

_C = 2048
_NBUF = 3


def _load(x_hbm, x_bufs, load_sems, j, s):
    return pltpu.make_async_copy(
        x_hbm.at[pl.ds(j * _C, _C), :],
        x_bufs.at[pl.ds(s * _C, _C), :],
        load_sems.at[s],
    )


def _store(o_bufs, out_hbm, store_sems, j, s):
    return pltpu.make_async_copy(
        o_bufs.at[pl.ds(s * _C, _C), :],
        out_hbm.at[pl.ds(j * _C, _C), :],
        store_sems.at[s],
    )


def _body(idx_hbm, w_hbm, x_hbm, out_hbm, x_bufs, o_bufs, idx_ref, w_ref,
          load_sems, store_sems, small_sem):
    n_rows = x_hbm.shape[0]
    n_chunks = n_rows // _C

    idx_cp = pltpu.make_async_copy(idx_hbm, idx_ref, small_sem)
    w_cp = pltpu.make_async_copy(w_hbm, w_ref, small_sem)
    idx_cp.start()
    w_cp.start()
    for j in range(_NBUF):
        _load(x_hbm, x_bufs, load_sems, j, j).start()
    idx_cp.wait()
    w_cp.wait()

    def step(i, carry):
        s = jax.lax.rem(i, _NBUF)
        _load(x_hbm, x_bufs, load_sems, i, s).wait()

        @pl.when(i >= _NBUF)
        def _():
            _store(o_bufs, out_hbm, store_sems, i - _NBUF, s).wait()

        scale = idx_ref[pl.ds(i, 1), :][0, :].astype(jnp.float32)[:, None]
        o_bufs[pl.ds(s * _C, _C), :] = (
            x_bufs[pl.ds(s * _C, _C), :] + scale * w_ref[...])

        @pl.when(i + _NBUF < n_chunks)
        def _():
            _load(x_hbm, x_bufs, load_sems, i + _NBUF, s).start()

        _store(o_bufs, out_hbm, store_sems, i, s).start()
        return carry

    jax.lax.fori_loop(0, n_chunks, step, 0)

    for j in range(n_chunks - _NBUF, n_chunks):
        _store(o_bufs, out_hbm, store_sems, j, j % _NBUF).wait()


def kernel(x, indices, embs_weight):
    B, S, D = x.shape
    n_rows = B * S
    n_chunks = n_rows // _C
    x2 = x.reshape(n_rows, D)
    idx2 = indices.reshape(n_chunks, _C)
    out = pl.pallas_call(
        _body,
        in_specs=[
            pl.BlockSpec(memory_space=pl.ANY),
            pl.BlockSpec(memory_space=pl.ANY),
            pl.BlockSpec(memory_space=pl.ANY),
        ],
        out_specs=pl.BlockSpec(memory_space=pl.ANY),
        out_shape=jax.ShapeDtypeStruct((n_rows, D), x.dtype),
        scratch_shapes=[
            pltpu.VMEM((_NBUF * _C, D), jnp.float32),
            pltpu.VMEM((_NBUF * _C, D), jnp.float32),
            pltpu.VMEM((n_chunks, _C), indices.dtype),
            pltpu.VMEM((1, D), jnp.float32),
            pltpu.SemaphoreType.DMA((_NBUF,)),
            pltpu.SemaphoreType.DMA((_NBUF,)),
            pltpu.SemaphoreType.DMA,
        ],
    )(idx2, embs_weight, x2)
    return out.reshape(B, S, D)
